# baseline (speedup 1.0000x reference)
import functools

import jax
import jax.numpy as jnp
from jax import lax
from jax.experimental import pallas as pl
from jax.experimental.pallas import tpu as pltpu
from jax.experimental.pallas import tpu_sc as plsc

B = 16384
NC = 2
NS = 16
NW = NC * NS
BPW = B // NW
NIDX = BPW // 128
NPASS = 2
PB = BPW // NPASS
PIDX = NIDX // NPASS


def _gather_wp_body(idw_h, idp_h, ew_h, ep_h, ow_h, op_h,
                    idw_v, idp_v, bw_v, bp_v, sw, sp):
    wid = lax.axis_index("s") * NC + lax.axis_index("c")
    base = wid * BPW
    pltpu.sync_copy(idw_h.at[wid], idw_v)
    pltpu.sync_copy(idp_h.at[wid], idp_v)
    for p in range(NPASS):
        ds = []
        for j in range(PIDX):
            r = p * PIDX + j
            ds.append(pltpu.async_copy(ew_h.at[idw_v.at[r]],
                                       bw_v.at[pl.ds(j * 128, 128)], sw))
            ds.append(pltpu.async_copy(ep_h.at[idp_v.at[r]],
                                       bp_v.at[pl.ds(j * 128, 128)], sp))
        for d in ds:
            d.wait()
        off = base + p * PB
        pltpu.sync_copy(bw_v, ow_h.at[pl.ds(off, PB)])
        pltpu.sync_copy(bp_v, op_h.at[pl.ds(off, PB)])


def _gather_a_body(ida_h, ea_h, oa_h, ida_v, ba_v, sa):
    wid = lax.axis_index("s") * NC + lax.axis_index("c")
    base = wid * BPW
    pltpu.sync_copy(ida_h.at[wid], ida_v)
    ds = []
    for j in range(NIDX):
        ds.append(pltpu.async_copy(ea_h.at[ida_v.at[j]],
                                   ba_v.at[pl.ds(j * 128, 128)], sa))
    for d in ds:
        d.wait()
    pltpu.sync_copy(ba_v, oa_h.at[pl.ds(base, BPW)])


_MESH = plsc.VectorSubcoreMesh(core_axis_name="c", subcore_axis_name="s")


def _sc_gather_wp(idw, idp, E_work, E_pub2):
    k = pl.kernel(
        _gather_wp_body,
        mesh=_MESH,
        out_type=[
            jax.ShapeDtypeStruct((B, 128), jnp.float32),
            jax.ShapeDtypeStruct((B, 128), jnp.float32),
        ],
        scratch_types=[
            pltpu.VMEM((NIDX, 128), jnp.int32),
            pltpu.VMEM((NIDX, 128), jnp.int32),
            pltpu.VMEM((PB, 128), jnp.float32),
            pltpu.VMEM((PB, 128), jnp.float32),
            pltpu.SemaphoreType.DMA,
            pltpu.SemaphoreType.DMA,
        ],
    )
    return k(idw, idp, E_work, E_pub2)


def _sc_gather_a(ida, E_auth2):
    k = pl.kernel(
        _gather_a_body,
        mesh=_MESH,
        out_type=jax.ShapeDtypeStruct((B, 128), jnp.float32),
        scratch_types=[
            pltpu.VMEM((NIDX, 128), jnp.int32),
            pltpu.VMEM((BPW, 128), jnp.float32),
            pltpu.SemaphoreType.DMA,
        ],
    )
    return k(ida, E_auth2)


RL = 2000


def _repack_body(x, o):
    t = jnp.transpose(x[...])
    o[...] = jnp.concatenate([t[0::2, :], t[1::2, :]], axis=1)


def _repack_auth(EaT):
    A = EaT.shape[1]
    return pl.pallas_call(
        _repack_body,
        grid=(A // RL,),
        in_specs=[pl.BlockSpec((64, RL), lambda i: (0, i))],
        out_specs=pl.BlockSpec((RL // 2, 128), lambda i: (i, 0)),
        out_shape=jax.ShapeDtypeStruct((A // 2, 128), jnp.float32),
    )(EaT)


BM = 2048
SUB = BM // 128


def _mlp_body(gw, ga2, gp2, pa, ti, ey,
              w0, b0, w1, b1, w2, b2, w3, b3, w4, b4, out):
    row = lax.broadcasted_iota(jnp.int32, (BM, 1), 0)
    oh_sub = (lax.broadcasted_iota(jnp.int32, (BM, SUB), 1)
              == row // 128).astype(jnp.float32)
    full = jnp.dot(oh_sub, pa[...], preferred_element_type=jnp.float32)
    lm = ((lax.broadcasted_iota(jnp.int32, (BM, 128), 1))
          == (row % 128)).astype(jnp.float32)
    yp_r = jnp.dot(full * lm, jnp.ones((128, 8), jnp.float32),
                   preferred_element_type=jnp.float32)[:, 0:1]

    bf = jnp.bfloat16

    def bdot(a, b):
        return jnp.dot(a.astype(bf), b.astype(bf),
                       preferred_element_type=jnp.float32)

    h = bdot(gw[...], w0[0:128, :])
    h += bdot(ga2[:, 0:64], w0[128:192, :])
    h += bdot(gp2[:, 0:64], w0[192:256, :])
    oh_y = (yp_r == lax.broadcasted_iota(jnp.int32, (BM, 32), 1).astype(jnp.float32))
    gy = jnp.dot(oh_y[:, 0:20].astype(jnp.float32), ey[...],
                 preferred_element_type=jnp.float32)
    h += bdot(gy, w0[256:288, :])
    h += bdot(ti[...], w0[288:672, :])
    h = jnp.maximum(h + b0[...], 0.0)
    h = jnp.maximum(bdot(h, w1[...]) + b1[...], 0.0)
    h = jnp.maximum(bdot(h, w2[...]) + b2[...], 0.0)
    h = jnp.maximum(bdot(h, w3[...]) + b3[...], 0.0)
    out[...] = bdot(h, w4[...]) + b4[...]


def _tc_mlp(gw, ga2, gp2, pa, title, E_yop,
            W0, b0, W1, b1, W2, b2, W3, b3, W4, b4):
    grid = (B // BM,)
    bs_row = lambda d: pl.BlockSpec((BM, d), lambda i: (i, 0))
    bs_sub = pl.BlockSpec((SUB, 128), lambda i: (i, 0))
    bs_full = lambda s: pl.BlockSpec(s, lambda i: tuple(0 for _ in s))
    return pl.pallas_call(
        _mlp_body,
        grid=grid,
        in_specs=[
            bs_row(128), bs_row(128), bs_row(128),
            bs_sub, bs_row(384),
            bs_full((20, 32)),
            bs_full((672, 256)), bs_full((1, 256)),
            bs_full((256, 256)), bs_full((1, 256)),
            bs_full((256, 256)), bs_full((1, 256)),
            bs_full((256, 256)), bs_full((1, 256)),
            bs_full((256, 128)), bs_full((1, 128)),
        ],
        out_specs=bs_row(128),
        out_shape=jax.ShapeDtypeStruct((B, 128), jnp.float32),
    )(gw, ga2, gp2, pa, title, E_yop,
      W0, b0.reshape(1, 256), W1, b1.reshape(1, 256), W2, b2.reshape(1, 256),
      W3, b3.reshape(1, 256), W4, b4.reshape(1, 128))


def kernel(work_id, author, publisher, yop_bin, title_embedding,
           E_work, E_auth, E_pub, E_yop,
           W0, b0, W1, b1, W2, b2, W3, b3, W4, b4):
    idw = work_id.reshape(NW, NIDX, 128)
    ida = author.reshape(NW, NIDX, 128)
    idp = publisher.reshape(NW, NIDX, 128)
    gw, gp2 = _sc_gather_wp(idw, idp, E_work,
                            jnp.pad(E_pub, ((0, 0), (0, 64))))
    ga2 = _sc_gather_a(ida, jnp.pad(E_auth, ((0, 0), (0, 64))))
    pa = yop_bin.astype(jnp.float32).reshape(B // 128, 128)
    return _tc_mlp(gw, ga2, gp2, pa, title_embedding.astype(jnp.bfloat16), E_yop,
                   W0, b0, W1, b1, W2, b2, W3, b3, W4, b4)


# speedup vs baseline: 1.0613x; 1.0613x over previous
import functools

import jax
import jax.numpy as jnp
from jax import lax
from jax.experimental import pallas as pl
from jax.experimental.pallas import tpu as pltpu
from jax.experimental.pallas import tpu_sc as plsc

B = 16384
NC = 2
NS = 16
NW = NC * NS
BPW = B // NW
NIDX = BPW // 128
NPASS = 2
PB = BPW // NPASS
PIDX = NIDX // NPASS


def _gather_wp_body(idw_h, idp_h, ew_h, ep_h, ow_h, op_h,
                    idw_v, idp_v, bw_v, bp_v, sw, sp):
    wid = lax.axis_index("s") * NC + lax.axis_index("c")
    base = wid * BPW
    pltpu.sync_copy(idw_h.at[wid], idw_v)
    pltpu.sync_copy(idp_h.at[wid], idp_v)
    for p in range(NPASS):
        ds = []
        for j in range(PIDX):
            r = p * PIDX + j
            ds.append(pltpu.async_copy(ew_h.at[idw_v.at[r]],
                                       bw_v.at[pl.ds(j * 128, 128)], sw))
            ds.append(pltpu.async_copy(ep_h.at[idp_v.at[r]],
                                       bp_v.at[pl.ds(j * 128, 128)], sp))
        for d in ds:
            d.wait()
        off = base + p * PB
        pltpu.sync_copy(bw_v, ow_h.at[pl.ds(off, PB)])
        pltpu.sync_copy(bp_v, op_h.at[pl.ds(off, PB)])


def _gather_a_body(ida_h, ea_h, oa_h, ida_v, ba_v, sa):
    wid = lax.axis_index("s") * NC + lax.axis_index("c")
    base = wid * BPW
    pltpu.sync_copy(ida_h.at[wid], ida_v)
    ds = []
    for j in range(NIDX):
        ds.append(pltpu.async_copy(ea_h.at[ida_v.at[j]],
                                   ba_v.at[pl.ds(j * 128, 128)], sa))
    for d in ds:
        d.wait()
    pltpu.sync_copy(ba_v, oa_h.at[pl.ds(base, BPW)])


_MESH = plsc.VectorSubcoreMesh(core_axis_name="c", subcore_axis_name="s")


def _sc_gather_wp(idw, idp, E_work, E_pub2):
    k = pl.kernel(
        _gather_wp_body,
        mesh=_MESH,
        out_type=[
            jax.ShapeDtypeStruct((B, 128), jnp.float32),
            jax.ShapeDtypeStruct((B, 128), jnp.float32),
        ],
        scratch_types=[
            pltpu.VMEM((NIDX, 128), jnp.int32),
            pltpu.VMEM((NIDX, 128), jnp.int32),
            pltpu.VMEM((PB, 128), jnp.float32),
            pltpu.VMEM((PB, 128), jnp.float32),
            pltpu.SemaphoreType.DMA,
            pltpu.SemaphoreType.DMA,
        ],
    )
    return k(idw, idp, E_work, E_pub2)


def _sc_gather_a(ida, E_auth2):
    k = pl.kernel(
        _gather_a_body,
        mesh=_MESH,
        out_type=jax.ShapeDtypeStruct((B, 128), jnp.float32),
        scratch_types=[
            pltpu.VMEM((NIDX, 128), jnp.int32),
            pltpu.VMEM((BPW, 128), jnp.float32),
            pltpu.SemaphoreType.DMA,
        ],
    )
    return k(ida, E_auth2)


RL = 2000


def _repack_body(x, o):
    t = jnp.transpose(x[...])
    o[...] = jnp.concatenate([t[0::2, :], t[1::2, :]], axis=1)


def _repack_auth(EaT):
    A = EaT.shape[1]
    return pl.pallas_call(
        _repack_body,
        grid=(A // RL,),
        in_specs=[pl.BlockSpec((64, RL), lambda i: (0, i))],
        out_specs=pl.BlockSpec((RL // 2, 128), lambda i: (i, 0)),
        out_shape=jax.ShapeDtypeStruct((A // 2, 128), jnp.float32),
    )(EaT)


BM = 4096
SUB = BM // 128


def _mlp_body(gw, ga2, gp2, pa, ti, ey,
              w0, b0, w1, b1, w2, b2, w3, b3, w4, b4, out):
    row = lax.broadcasted_iota(jnp.int32, (BM, 1), 0)
    oh_sub = (lax.broadcasted_iota(jnp.int32, (BM, SUB), 1)
              == row // 128).astype(jnp.float32)
    full = jnp.dot(oh_sub, pa[...], preferred_element_type=jnp.float32)
    lm = ((lax.broadcasted_iota(jnp.int32, (BM, 128), 1))
          == (row % 128)).astype(jnp.float32)
    yp_r = jnp.dot(full * lm, jnp.ones((128, 8), jnp.float32),
                   preferred_element_type=jnp.float32)[:, 0:1]

    bf = jnp.bfloat16

    def bdot(a, b):
        return jnp.dot(a.astype(bf), b.astype(bf),
                       preferred_element_type=jnp.float32)

    h = bdot(gw[...], w0[0:128, :])
    h += bdot(ga2[:, 0:64], w0[128:192, :])
    h += bdot(gp2[:, 0:64], w0[192:256, :])
    oh_y = (yp_r == lax.broadcasted_iota(jnp.int32, (BM, 32), 1).astype(jnp.float32))
    gy = jnp.dot(oh_y[:, 0:20].astype(jnp.float32), ey[...],
                 preferred_element_type=jnp.float32)
    h += bdot(gy, w0[256:288, :])
    h += bdot(ti[...], w0[288:672, :])
    h = jnp.maximum(h + b0[...], 0.0)
    h = jnp.maximum(bdot(h, w1[...]) + b1[...], 0.0)
    h = jnp.maximum(bdot(h, w2[...]) + b2[...], 0.0)
    h = jnp.maximum(bdot(h, w3[...]) + b3[...], 0.0)
    out[...] = bdot(h, w4[...]) + b4[...]


def _tc_mlp(gw, ga2, gp2, pa, title, E_yop,
            W0, b0, W1, b1, W2, b2, W3, b3, W4, b4):
    grid = (B // BM,)
    bs_row = lambda d: pl.BlockSpec((BM, d), lambda i: (i, 0))
    bs_sub = pl.BlockSpec((SUB, 128), lambda i: (i, 0))
    bs_full = lambda s: pl.BlockSpec(s, lambda i: tuple(0 for _ in s))
    return pl.pallas_call(
        _mlp_body,
        grid=grid,
        in_specs=[
            bs_row(128), bs_row(128), bs_row(128),
            bs_sub, bs_row(384),
            bs_full((20, 32)),
            bs_full((672, 256)), bs_full((1, 256)),
            bs_full((256, 256)), bs_full((1, 256)),
            bs_full((256, 256)), bs_full((1, 256)),
            bs_full((256, 256)), bs_full((1, 256)),
            bs_full((256, 128)), bs_full((1, 128)),
        ],
        out_specs=bs_row(128),
        out_shape=jax.ShapeDtypeStruct((B, 128), jnp.float32),
    )(gw, ga2, gp2, pa, title, E_yop,
      W0, b0.reshape(1, 256), W1, b1.reshape(1, 256), W2, b2.reshape(1, 256),
      W3, b3.reshape(1, 256), W4, b4.reshape(1, 128))


def kernel(work_id, author, publisher, yop_bin, title_embedding,
           E_work, E_auth, E_pub, E_yop,
           W0, b0, W1, b1, W2, b2, W3, b3, W4, b4):
    idw = work_id.reshape(NW, NIDX, 128)
    ida = author.reshape(NW, NIDX, 128)
    idp = publisher.reshape(NW, NIDX, 128)
    gw, gp2 = _sc_gather_wp(idw, idp, E_work,
                            jnp.pad(E_pub, ((0, 0), (0, 64))))
    ga2 = _sc_gather_a(ida, jnp.pad(E_auth, ((0, 0), (0, 64))))
    pa = yop_bin.astype(jnp.float32).reshape(B // 128, 128)
    return _tc_mlp(gw, ga2, gp2, pa, title_embedding, E_yop,
                   W0, b0, W1, b1, W2, b2, W3, b3, W4, b4)


# speedup vs baseline: 1.1737x; 1.1059x over previous
import functools

import jax
import jax.numpy as jnp
from jax import lax
from jax.experimental import pallas as pl
from jax.experimental.pallas import tpu as pltpu
from jax.experimental.pallas import tpu_sc as plsc

B = 16384
NC = 2
NS = 16
NW = NC * NS
BPW = B // NW
NIDX = BPW // 128
NPASS = 2
PB = BPW // NPASS
PIDX = NIDX // NPASS


def _gather_wp_body(idw_h, idp_h, ew_h, ep_h, ow_h, op_h,
                    idw_v, idp_v, bw_v, bp_v, sw, sp):
    wid = lax.axis_index("s") * NC + lax.axis_index("c")
    base = wid * BPW
    pltpu.sync_copy(idw_h.at[wid], idw_v)
    pltpu.sync_copy(idp_h.at[wid], idp_v)
    for p in range(NPASS):
        ds = []
        for j in range(PIDX):
            r = p * PIDX + j
            ds.append(pltpu.async_copy(ew_h.at[idw_v.at[r]],
                                       bw_v.at[pl.ds(j * 128, 128)], sw))
            ds.append(pltpu.async_copy(ep_h.at[idp_v.at[r]],
                                       bp_v.at[pl.ds(j * 128, 128)], sp))
        for d in ds:
            d.wait()
        off = base + p * PB
        pltpu.sync_copy(bw_v, ow_h.at[pl.ds(off, PB)])
        pltpu.sync_copy(bp_v, op_h.at[pl.ds(off, PB)])


def _gather_a_body(ida_h, ea_h, oa_h, ida_v, ba_v, sa):
    wid = lax.axis_index("s") * NC + lax.axis_index("c")
    base = wid * BPW
    pltpu.sync_copy(ida_h.at[wid], ida_v)
    ds = []
    for j in range(NIDX):
        ds.append(pltpu.async_copy(ea_h.at[ida_v.at[j]],
                                   ba_v.at[pl.ds(j * 128, 128)], sa))
    for d in ds:
        d.wait()
    pltpu.sync_copy(ba_v, oa_h.at[pl.ds(base, BPW)])


_MESH = plsc.VectorSubcoreMesh(core_axis_name="c", subcore_axis_name="s")


def _sc_gather_wp(idw, idp, E_work, E_pub2):
    k = pl.kernel(
        _gather_wp_body,
        mesh=_MESH,
        out_type=[
            jax.ShapeDtypeStruct((B, 128), jnp.float32),
            jax.ShapeDtypeStruct((B, 128), jnp.float32),
        ],
        scratch_types=[
            pltpu.VMEM((NIDX, 128), jnp.int32),
            pltpu.VMEM((NIDX, 128), jnp.int32),
            pltpu.VMEM((PB, 128), jnp.float32),
            pltpu.VMEM((PB, 128), jnp.float32),
            pltpu.SemaphoreType.DMA,
            pltpu.SemaphoreType.DMA,
        ],
    )
    return k(idw, idp, E_work, E_pub2)


def _sc_gather_a(ida, E_auth2):
    k = pl.kernel(
        _gather_a_body,
        mesh=_MESH,
        out_type=jax.ShapeDtypeStruct((B, 128), jnp.float32),
        scratch_types=[
            pltpu.VMEM((NIDX, 128), jnp.int32),
            pltpu.VMEM((BPW, 128), jnp.float32),
            pltpu.SemaphoreType.DMA,
        ],
    )
    return k(ida, E_auth2)


RL = 2000


def _repack_body(x, o):
    t = jnp.transpose(x[...])
    o[...] = jnp.concatenate([t[0::2, :], t[1::2, :]], axis=1)


def _repack_auth(EaT):
    A = EaT.shape[1]
    return pl.pallas_call(
        _repack_body,
        grid=(A // RL,),
        in_specs=[pl.BlockSpec((64, RL), lambda i: (0, i))],
        out_specs=pl.BlockSpec((RL // 2, 128), lambda i: (i, 0)),
        out_shape=jax.ShapeDtypeStruct((A // 2, 128), jnp.float32),
    )(EaT)


BM = 2048
SUB = BM // 128


def _mlp_body(gw, ga2, gp2, pa, ti, ey,
              w0, b0, w1, b1, w2, b2, w3, b3, w4, b4, out):
    row = lax.broadcasted_iota(jnp.int32, (BM, 1), 0)
    oh_sub = (lax.broadcasted_iota(jnp.int32, (BM, SUB), 1)
              == row // 128).astype(jnp.float32)
    full = jnp.dot(oh_sub, pa[...], preferred_element_type=jnp.float32)
    lm = ((lax.broadcasted_iota(jnp.int32, (BM, 128), 1))
          == (row % 128)).astype(jnp.float32)
    yp_r = jnp.dot(full * lm, jnp.ones((128, 8), jnp.float32),
                   preferred_element_type=jnp.float32)[:, 0:1]

    bf = jnp.bfloat16

    def bdot(a, b):
        return jnp.dot(a.astype(bf), b.astype(bf),
                       preferred_element_type=jnp.float32)

    oh_y = (yp_r == lax.broadcasted_iota(jnp.int32, (BM, 32), 1).astype(jnp.float32))
    gy = jnp.dot(oh_y[:, 0:20].astype(jnp.float32), ey[...],
                 preferred_element_type=jnp.float32)
    x = jnp.concatenate([gw[...], ga2[:, 0:64], gp2[:, 0:64], gy, ti[...]],
                        axis=1)
    h = jnp.maximum(bdot(x, w0[...]) + b0[...], 0.0)
    h = jnp.maximum(bdot(h, w1[...]) + b1[...], 0.0)
    h = jnp.maximum(bdot(h, w2[...]) + b2[...], 0.0)
    h = jnp.maximum(bdot(h, w3[...]) + b3[...], 0.0)
    out[...] = bdot(h, w4[...]) + b4[...]


def _tc_mlp(gw, ga2, gp2, pa, title, E_yop,
            W0, b0, W1, b1, W2, b2, W3, b3, W4, b4):
    grid = (B // BM,)
    bs_row = lambda d: pl.BlockSpec((BM, d), lambda i: (i, 0))
    bs_sub = pl.BlockSpec((SUB, 128), lambda i: (i, 0))
    bs_full = lambda s: pl.BlockSpec(s, lambda i: tuple(0 for _ in s))
    return pl.pallas_call(
        _mlp_body,
        grid=grid,
        in_specs=[
            bs_row(128), bs_row(128), bs_row(128),
            bs_sub, bs_row(384),
            bs_full((20, 32)),
            bs_full((672, 256)), bs_full((1, 256)),
            bs_full((256, 256)), bs_full((1, 256)),
            bs_full((256, 256)), bs_full((1, 256)),
            bs_full((256, 256)), bs_full((1, 256)),
            bs_full((256, 128)), bs_full((1, 128)),
        ],
        out_specs=bs_row(128),
        out_shape=jax.ShapeDtypeStruct((B, 128), jnp.float32),
    )(gw, ga2, gp2, pa, title, E_yop,
      W0, b0.reshape(1, 256), W1, b1.reshape(1, 256), W2, b2.reshape(1, 256),
      W3, b3.reshape(1, 256), W4, b4.reshape(1, 128))


def kernel(work_id, author, publisher, yop_bin, title_embedding,
           E_work, E_auth, E_pub, E_yop,
           W0, b0, W1, b1, W2, b2, W3, b3, W4, b4):
    idw = work_id.reshape(NW, NIDX, 128)
    ida = author.reshape(NW, NIDX, 128)
    idp = publisher.reshape(NW, NIDX, 128)
    gw, gp2 = _sc_gather_wp(idw, idp, E_work,
                            jnp.pad(E_pub, ((0, 0), (0, 64))))
    ga2 = _sc_gather_a(ida, jnp.pad(E_auth, ((0, 0), (0, 64))))
    pa = yop_bin.astype(jnp.float32).reshape(B // 128, 128)
    return _tc_mlp(gw, ga2, gp2, pa, title_embedding, E_yop,
                   W0, b0, W1, b1, W2, b2, W3, b3, W4, b4)
